# baseline (speedup 1.0000x reference)
import functools

import jax
import jax.numpy as jnp
from jax import lax
from jax.experimental import pallas as pl
from jax.experimental.pallas import tpu as pltpu
from jax.experimental.pallas import tpu_sc as plsc

V = 1000000
E = 32
B = 16384

_PK = 128 // E
_TPC = 32768
_NBLK = pl.cdiv(V, _TPC)
_VQ = _NBLK * (_TPC // _PK)
_VB = _NBLK * (_TPC // 128)

_NC = 2
_NS = 16
_NW = _NC * _NS
_CHUNK = B // _NW
_IDXC = 64
_NCHUNK = _CHUNK // _IDXC
_L = 16
_NGROUP = _IDXC // _L

_SH_BLK = _TPC.bit_length() - 1
_SH_Q = (_TPC // _PK).bit_length() - 1


def _tp_body(x_ref, b_ref, o_ref, ob_ref):
    x = x_ref[...].astype(jnp.bfloat16)
    ident = jnp.eye(E, dtype=jnp.bfloat16)
    t = lax.dot_general(x, ident, (((0,), (0,)), ((), ())),
                        preferred_element_type=jnp.float32)
    q = _TPC // _PK
    for r in range(_PK):
        o_ref[:, r * E:(r + 1) * E] = t[r * q:(r + 1) * q, :]
    for l in range(_TPC // 128):
        ob_ref[l, :] = b_ref[0, l * 128:(l + 1) * 128]


def _transpose_table(tT, biasT):
    return pl.pallas_call(
        _tp_body,
        grid=(_NBLK,),
        in_specs=[pl.BlockSpec((E, _TPC), lambda i: (0, i)),
                  pl.BlockSpec((1, _TPC), lambda i: (0, i))],
        out_specs=[pl.BlockSpec((_TPC // _PK, _PK * E), lambda i: (i, 0)),
                   pl.BlockSpec((_TPC // 128, 128), lambda i: (i, 0))],
        out_shape=[jax.ShapeDtypeStruct((_VQ, _PK * E), jnp.float32),
                   jax.ShapeDtypeStruct((_VB, 128), jnp.float32)],
    )(tT, biasT)


def _glove_body(center_hbm, outside_hbm, coocs_hbm, wt_hbm,
                cemb_hbm, oemb_hbm, cbias_hbm, obias_hbm, out_hbm,
                idx_c, idx_o, qidx_c, qidx_o, bidx_c, bidx_o,
                buf_c0, buf_o0, buf_c1, buf_o1, bufb_c, bufb_o,
                cooc_v, wt_v, acc_v, sem0, sem1, bsem):
    bufs = [(buf_c0, buf_o0, sem0), (buf_c1, buf_o1, sem1)]
    wid = lax.axis_index("s") * _NC + lax.axis_index("c")

    pltpu.sync_copy(center_hbm.at[wid], idx_c)
    pltpu.sync_copy(outside_hbm.at[wid], idx_o)
    pltpu.sync_copy(coocs_hbm.at[wid], cooc_v)
    pltpu.sync_copy(wt_hbm.at[wid], wt_v)

    for j in range(_NCHUNK):
        for s in range(_IDXC // _L):
            sl = pl.ds(s * _L, _L)
            vc = idx_c.at[j][sl]
            vo = idx_o.at[j][sl]
            qidx_c.at[j][sl] = (
                lax.shift_left(lax.shift_right_logical(vc, _SH_BLK), _SH_Q)
                + (vc & (_TPC // _PK - 1)))
            qidx_o.at[j][sl] = (
                lax.shift_left(lax.shift_right_logical(vo, _SH_BLK), _SH_Q)
                + (vo & (_TPC // _PK - 1)))
            bidx_c.at[j][sl] = lax.shift_right_logical(vc, 7)
            bidx_o.at[j][sl] = lax.shift_right_logical(vo, 7)

    lanes = lax.iota(jnp.int32, _L)
    acc = jnp.zeros((_L,), jnp.float32)

    def fire(j):
        bc_, bo_, s = bufs[j % 2]
        return [pltpu.async_copy(cemb_hbm.at[qidx_c.at[j]], bc_, s),
                pltpu.async_copy(oemb_hbm.at[qidx_o.at[j]], bo_, s)]

    def fire_bias(j):
        return [pltpu.async_copy(cbias_hbm.at[bidx_c.at[j]], bufb_c, bsem),
                pltpu.async_copy(obias_hbm.at[bidx_o.at[j]], bufb_o, bsem)]

    pending = {0: fire(0)}
    bias_pending = fire_bias(0)
    for j in range(_NCHUNK):
        if j + 1 < _NCHUNK:
            pending[j + 1] = fire(j + 1)
        for h in pending.pop(j) + bias_pending:
            h.wait()
        buf_c, buf_o, _ = bufs[j % 2]
        for g in range(_NGROUP):
            sl = pl.ds(g * _L, _L)
            rows = g * _L + lanes
            vc = idx_c.at[j][sl]
            vo = idx_o.at[j][sl]
            colc = (lax.shift_right_logical(vc, _SH_Q) & (_PK - 1)) * E
            colo = (lax.shift_right_logical(vo, _SH_Q) & (_PK - 1)) * E
            ip = jnp.zeros((_L,), jnp.float32)
            for e in range(E):
                cv = plsc.load_gather(buf_c, [rows, colc + e])
                ov = plsc.load_gather(buf_o, [rows, colo + e])
                ip = ip + cv * ov
            cb = plsc.load_gather(bufb_c, [rows, vc & 127])
            tb = plsc.load_gather(bufb_o, [rows, vo & 127])
            psl = pl.ds(j * _IDXC + g * _L, _L)
            d = ip + cb + tb - cooc_v[psl]
            acc = acc + wt_v[psl] * d * d
        if j + 1 < _NCHUNK:
            bias_pending = fire_bias(j + 1)

    acc_v[...] = acc
    pltpu.sync_copy(acc_v, out_hbm.at[wid])


@jax.jit
def _glove(center, outside, coocs, weighting,
           cembT, oembT, cbiasT, obiasT):
    cemb_q, cbias_q = _transpose_table(cembT, cbiasT)
    oemb_q, obias_q = _transpose_table(oembT, obiasT)
    kern = functools.partial(
        pl.kernel,
        mesh=plsc.VectorSubcoreMesh(core_axis_name="c", subcore_axis_name="s"),
        out_type=jax.ShapeDtypeStruct((_NW, _L), jnp.float32),
        compiler_params=pltpu.CompilerParams(needs_layout_passes=False),
        scratch_types=[
            pltpu.VMEM((_NCHUNK, _IDXC), jnp.int32),
            pltpu.VMEM((_NCHUNK, _IDXC), jnp.int32),
            pltpu.VMEM((_NCHUNK, _IDXC), jnp.int32),
            pltpu.VMEM((_NCHUNK, _IDXC), jnp.int32),
            pltpu.VMEM((_NCHUNK, _IDXC), jnp.int32),
            pltpu.VMEM((_NCHUNK, _IDXC), jnp.int32),
            pltpu.VMEM((_IDXC, _PK * E), jnp.float32),
            pltpu.VMEM((_IDXC, _PK * E), jnp.float32),
            pltpu.VMEM((_IDXC, _PK * E), jnp.float32),
            pltpu.VMEM((_IDXC, _PK * E), jnp.float32),
            pltpu.VMEM((_IDXC, 128), jnp.float32),
            pltpu.VMEM((_IDXC, 128), jnp.float32),
            pltpu.VMEM((_CHUNK,), jnp.float32),
            pltpu.VMEM((_CHUNK,), jnp.float32),
            pltpu.VMEM((_L,), jnp.float32),
            pltpu.SemaphoreType.DMA,
            pltpu.SemaphoreType.DMA,
            pltpu.SemaphoreType.DMA,
        ],
    )(_glove_body)
    partials = kern(center, outside, coocs, weighting,
                    cemb_q, oemb_q, cbias_q, obias_q)
    return jnp.sum(partials)


def kernel(center, outside, coocs, weighting,
           center_emb, outside_emb, center_bias, outside_bias):
    center = center.reshape(_NW, _NCHUNK, _IDXC).astype(jnp.int32)
    outside = outside.reshape(_NW, _NCHUNK, _IDXC).astype(jnp.int32)
    coocs = coocs.reshape(_NW, _CHUNK)
    weighting = weighting.reshape(_NW, _CHUNK)
    return _glove(center, outside, coocs, weighting,
                  center_emb.T, outside_emb.T,
                  center_bias.T, outside_bias.T)


# speedup vs baseline: 1.0184x; 1.0184x over previous
import functools

import jax
import jax.numpy as jnp
from jax import lax
from jax.experimental import pallas as pl
from jax.experimental.pallas import tpu as pltpu
from jax.experimental.pallas import tpu_sc as plsc

V = 1000000
E = 32
B = 16384

_PK = 128 // E
_TPC = 32768
_NBLK = pl.cdiv(V, _TPC)
_VQ = _NBLK * (_TPC // _PK)
_VB = _NBLK * (_TPC // 128)

_NC = 2
_NS = 16
_NW = _NC * _NS
_CHUNK = B // _NW
_IDXC = 128
_NCHUNK = _CHUNK // _IDXC
_L = 16
_NGROUP = _IDXC // _L

_SH_BLK = _TPC.bit_length() - 1
_SH_Q = (_TPC // _PK).bit_length() - 1


def _tp_body(x_ref, b_ref, o_ref, ob_ref):
    x = x_ref[...].astype(jnp.bfloat16)
    ident = jnp.eye(E, dtype=jnp.bfloat16)
    t = lax.dot_general(x, ident, (((0,), (0,)), ((), ())),
                        preferred_element_type=jnp.float32)
    q = _TPC // _PK
    for r in range(_PK):
        o_ref[:, r * E:(r + 1) * E] = t[r * q:(r + 1) * q, :]
    for l in range(_TPC // 128):
        ob_ref[l, :] = b_ref[0, l * 128:(l + 1) * 128]


def _transpose_table(tT, biasT):
    return pl.pallas_call(
        _tp_body,
        grid=(_NBLK,),
        in_specs=[pl.BlockSpec((E, _TPC), lambda i: (0, i)),
                  pl.BlockSpec((1, _TPC), lambda i: (0, i))],
        out_specs=[pl.BlockSpec((_TPC // _PK, _PK * E), lambda i: (i, 0)),
                   pl.BlockSpec((_TPC // 128, 128), lambda i: (i, 0))],
        out_shape=[jax.ShapeDtypeStruct((_VQ, _PK * E), jnp.float32),
                   jax.ShapeDtypeStruct((_VB, 128), jnp.float32)],
    )(tT, biasT)


def _glove_body(center_hbm, outside_hbm, coocs_hbm, wt_hbm,
                cemb_hbm, oemb_hbm, cbias_hbm, obias_hbm, out_hbm,
                idx_c, idx_o, qidx_c, qidx_o, bidx_c, bidx_o,
                buf_c, buf_o, bufb_c, bufb_o,
                cooc_v, wt_v, acc_v, sem):
    wid = lax.axis_index("s") * _NC + lax.axis_index("c")

    pltpu.sync_copy(center_hbm.at[wid], idx_c)
    pltpu.sync_copy(outside_hbm.at[wid], idx_o)
    pltpu.sync_copy(coocs_hbm.at[wid], cooc_v)
    pltpu.sync_copy(wt_hbm.at[wid], wt_v)

    for j in range(_NCHUNK):
        for s in range(_IDXC // _L):
            sl = pl.ds(s * _L, _L)
            vc = idx_c.at[j][sl]
            vo = idx_o.at[j][sl]
            qidx_c.at[j][sl] = (
                lax.shift_left(lax.shift_right_logical(vc, _SH_BLK), _SH_Q)
                + (vc & (_TPC // _PK - 1)))
            qidx_o.at[j][sl] = (
                lax.shift_left(lax.shift_right_logical(vo, _SH_BLK), _SH_Q)
                + (vo & (_TPC // _PK - 1)))
            bidx_c.at[j][sl] = lax.shift_right_logical(vc, 7)
            bidx_o.at[j][sl] = lax.shift_right_logical(vo, 7)

    lanes = lax.iota(jnp.int32, _L)
    acc = jnp.zeros((_L,), jnp.float32)

    for j in range(_NCHUNK):
        cc = pltpu.async_copy(cemb_hbm.at[qidx_c.at[j]], buf_c, sem)
        oc = pltpu.async_copy(oemb_hbm.at[qidx_o.at[j]], buf_o, sem)
        bc = pltpu.async_copy(cbias_hbm.at[bidx_c.at[j]], bufb_c, sem)
        bo = pltpu.async_copy(obias_hbm.at[bidx_o.at[j]], bufb_o, sem)
        cc.wait()
        oc.wait()
        bc.wait()
        bo.wait()
        for g in range(_NGROUP):
            sl = pl.ds(g * _L, _L)
            rows = g * _L + lanes
            vc = idx_c.at[j][sl]
            vo = idx_o.at[j][sl]
            colc = (lax.shift_right_logical(vc, _SH_Q) & (_PK - 1)) * E
            colo = (lax.shift_right_logical(vo, _SH_Q) & (_PK - 1)) * E
            ip = jnp.zeros((_L,), jnp.float32)
            for e in range(E):
                cv = plsc.load_gather(buf_c, [rows, colc + e])
                ov = plsc.load_gather(buf_o, [rows, colo + e])
                ip = ip + cv * ov
            cb = plsc.load_gather(bufb_c, [rows, vc & 127])
            tb = plsc.load_gather(bufb_o, [rows, vo & 127])
            psl = pl.ds(j * _IDXC + g * _L, _L)
            d = ip + cb + tb - cooc_v[psl]
            acc = acc + wt_v[psl] * d * d

    acc_v[...] = acc
    pltpu.sync_copy(acc_v, out_hbm.at[wid])


@jax.jit
def _glove(center, outside, coocs, weighting,
           cembT, oembT, cbiasT, obiasT):
    cemb_q, cbias_q = _transpose_table(cembT, cbiasT)
    oemb_q, obias_q = _transpose_table(oembT, obiasT)
    kern = functools.partial(
        pl.kernel,
        mesh=plsc.VectorSubcoreMesh(core_axis_name="c", subcore_axis_name="s"),
        out_type=jax.ShapeDtypeStruct((_NW, _L), jnp.float32),
        compiler_params=pltpu.CompilerParams(needs_layout_passes=False),
        scratch_types=[
            pltpu.VMEM((_NCHUNK, _IDXC), jnp.int32),
            pltpu.VMEM((_NCHUNK, _IDXC), jnp.int32),
            pltpu.VMEM((_NCHUNK, _IDXC), jnp.int32),
            pltpu.VMEM((_NCHUNK, _IDXC), jnp.int32),
            pltpu.VMEM((_NCHUNK, _IDXC), jnp.int32),
            pltpu.VMEM((_NCHUNK, _IDXC), jnp.int32),
            pltpu.VMEM((_IDXC, _PK * E), jnp.float32),
            pltpu.VMEM((_IDXC, _PK * E), jnp.float32),
            pltpu.VMEM((_IDXC, 128), jnp.float32),
            pltpu.VMEM((_IDXC, 128), jnp.float32),
            pltpu.VMEM((_CHUNK,), jnp.float32),
            pltpu.VMEM((_CHUNK,), jnp.float32),
            pltpu.VMEM((_L,), jnp.float32),
            pltpu.SemaphoreType.DMA,
        ],
    )(_glove_body)
    partials = kern(center, outside, coocs, weighting,
                    cemb_q, oemb_q, cbias_q, obias_q)
    return jnp.sum(partials)


def kernel(center, outside, coocs, weighting,
           center_emb, outside_emb, center_bias, outside_bias):
    center = center.reshape(_NW, _NCHUNK, _IDXC).astype(jnp.int32)
    outside = outside.reshape(_NW, _NCHUNK, _IDXC).astype(jnp.int32)
    coocs = coocs.reshape(_NW, _CHUNK)
    weighting = weighting.reshape(_NW, _CHUNK)
    return _glove(center, outside, coocs, weighting,
                  center_emb.T, outside_emb.T,
                  center_bias.T, outside_bias.T)
